# Initial kernel scaffold; baseline (speedup 1.0000x reference)
import jax
import jax.numpy as jnp
from jax.experimental import pallas as pl


def kernel(token_ids, weight):
    raise NotImplementedError("write your pallas kernel here")


# speedup vs baseline: 3.4490x; 3.4490x over previous
import functools

import jax
import jax.numpy as jnp
from jax import lax
from jax.experimental import pallas as pl
from jax.experimental.pallas import tpu as pltpu
from jax.experimental.pallas import tpu_sc as plsc

_D = 64
_NC = 2
_NS = 16
_NW = _NC * _NS
_CHUNK = 800


@functools.partial(jax.jit, static_argnames=("total",))
def _gather(weight, idx, total):
    b_per_w = total // _NW
    n_chunks = b_per_w // _CHUNK
    mesh = plsc.VectorSubcoreMesh(core_axis_name="c", subcore_axis_name="s")

    @functools.partial(
        pl.kernel,
        mesh=mesh,
        out_type=jax.ShapeDtypeStruct((total, _D), jnp.float32),
        scratch_types=[
            pltpu.VMEM((_CHUNK,), jnp.int32),
            pltpu.VMEM((_CHUNK, _D), jnp.float32),
            pltpu.SemaphoreType.DMA,
        ],
        compiler_params=pltpu.CompilerParams(use_tc_tiling_on_sc=False),
    )
    def gather_kernel(table_hbm, idx_hbm, out_hbm, idx_v, rows_v, sem):
        wid = lax.axis_index("s") * _NC + lax.axis_index("c")
        base = wid * b_per_w

        def body(c, carry):
            off = base + c * _CHUNK
            pltpu.sync_copy(idx_hbm.at[pl.ds(off, _CHUNK)], idx_v)
            pltpu.async_copy(table_hbm.at[idx_v], rows_v, sem).wait()
            pltpu.sync_copy(rows_v, out_hbm.at[pl.ds(off, _CHUNK)])
            return carry

        lax.fori_loop(0, n_chunks, body, 0)

    return gather_kernel(weight, idx)


def kernel(token_ids, weight):
    shape = token_ids.shape
    idx = token_ids.reshape(-1).astype(jnp.int32)
    out = _gather(weight, idx, idx.shape[0])
    return out.reshape(*shape, _D)


# speedup vs baseline: 3.4969x; 1.0139x over previous
import functools

import jax
import jax.numpy as jnp
from jax import lax
from jax.experimental import pallas as pl
from jax.experimental.pallas import tpu as pltpu
from jax.experimental.pallas import tpu_sc as plsc

_D = 64
_NC = 2
_NS = 16
_NW = _NC * _NS
_CHUNK = 800


@functools.partial(jax.jit, static_argnames=("total",))
def _gather(weight, idx, total):
    b_per_w = total // _NW
    n_chunks = b_per_w // _CHUNK
    mesh = plsc.VectorSubcoreMesh(core_axis_name="c", subcore_axis_name="s")

    @functools.partial(
        pl.kernel,
        mesh=mesh,
        out_type=jax.ShapeDtypeStruct((total, _D), jnp.float32),
        scratch_types=[
            pltpu.VMEM((2, _CHUNK), jnp.int32),
            pltpu.VMEM((2, _CHUNK, _D), jnp.float32),
            pltpu.SemaphoreType.DMA,
            pltpu.SemaphoreType.DMA,
            pltpu.SemaphoreType.DMA,
        ],
        compiler_params=pltpu.CompilerParams(use_tc_tiling_on_sc=False),
    )
    def gather_kernel(table_hbm, idx_hbm, out_hbm, idx_v, rows_v,
                      sem_i, sem_g, sem_o):
        wid = lax.axis_index("s") * _NC + lax.axis_index("c")
        base = wid * b_per_w

        def idx_load(c, b):
            off = base + c * _CHUNK
            return pltpu.async_copy(idx_hbm.at[pl.ds(off, _CHUNK)],
                                    idx_v.at[b], sem_i)

        def gather(b):
            return pltpu.async_copy(table_hbm.at[idx_v.at[b]],
                                    rows_v.at[b], sem_g)

        def writeback(c, b):
            off = base + c * _CHUNK
            return pltpu.async_copy(rows_v.at[b],
                                    out_hbm.at[pl.ds(off, _CHUNK)], sem_o)

        d_i = [None] * n_chunks
        d_g = [None] * n_chunks
        d_o = [None] * n_chunks
        d_i[0] = idx_load(0, 0)
        d_i[0].wait()
        d_g[0] = gather(0)
        if n_chunks > 1:
            d_i[1] = idx_load(1, 1)
        for c in range(n_chunks):
            b = c % 2
            nb = (c + 1) % 2
            d_g[c].wait()
            if c + 1 < n_chunks:
                d_i[c + 1].wait()
                if c >= 1:
                    d_o[c - 1].wait()
                d_g[c + 1] = gather(nb)
                if c + 2 < n_chunks:
                    d_i[c + 2] = idx_load(c + 2, b)
            d_o[c] = writeback(c, b)
        d_o[n_chunks - 1].wait()
        if n_chunks >= 2:
            d_o[n_chunks - 2].wait()

    return gather_kernel(weight, idx)


def kernel(token_ids, weight):
    shape = token_ids.shape
    idx = token_ids.reshape(-1).astype(jnp.int32)
    out = _gather(weight, idx, idx.shape[0])
    return out.reshape(*shape, _D)
